# baseline (speedup 1.0000x reference)
import functools
import math

import jax
import jax.numpy as jnp
from jax import lax
from jax.experimental import pallas as pl
from jax.experimental.pallas import tpu as pltpu
from jax.experimental.pallas import tpu_sc as plsc

C = 256
NQ = 10
NH = 8
DH = C // NH
FF = 512
NEG = float("-inf")


def _ln_rows(x, g, b):
    m = jnp.mean(x, axis=-1, keepdims=True)
    v = jnp.mean((x - m) ** 2, axis=-1, keepdims=True)
    return (x - m) / jnp.sqrt(v + 1e-5) * g + b


def _nn(a, b):
    return jax.lax.dot_general(a, b, (((1,), (0,)), ((), ())),
                               preferred_element_type=jnp.float32)


def _nt(a, b):
    return jax.lax.dot_general(a, b, (((1,), (1,)), ((), ())),
                               preferred_element_type=jnp.float32)


def _precompute_body(H, W, sa_Wo, sa_bo, sa_bv, n1g, n1b,
                     ca_Wv, ca_Wo, ca_bv, ca_bo,
                     posF_ref, M_ref, vec_ref):
    HW = H * W
    ci = jax.lax.broadcasted_iota(jnp.int32, (HW, C), 1)
    pi = jax.lax.broadcasted_iota(jnp.int32, (HW, C), 0)
    i = pi // W
    j = pi % W
    scale = 2.0 * math.pi
    yv = (i.astype(jnp.float32) + 1.0) / (H + 1e-6) * scale
    xv = (j.astype(jnp.float32) + 1.0) / (W + 1e-6) * scale
    k = (ci % (C // 2)) // 2
    tw = jnp.exp(k.astype(jnp.float32) * (2.0 / (C // 2)) * math.log(10000.0))
    val = jnp.where(ci < (C // 2), yv, xv) / tw
    posF_ref[...] = jnp.where(ci % 2 == 0, jnp.sin(val), jnp.cos(val))
    for h in range(NH):
        M_ref[h] = _nn(ca_Wv[:, h * DH:(h + 1) * DH],
                       ca_Wo[h * DH:(h + 1) * DH, :])
    c0 = _nn(sa_bv[...], sa_Wo[...]) + sa_bo[...]
    q1 = _ln_rows(c0, n1g[...], n1b[...])
    cb = _nn(ca_bv[...], ca_Wo[...]) + ca_bo[...]
    vec_ref[...] = jnp.concatenate(
        [q1, cb, jnp.zeros((6, C), jnp.float32)], axis=0)


def _sc_text_topk(text_feat):
    B, L, Cc = text_feat.shape
    info = plsc.get_sparse_core_info()
    ncores, nsub, LN = info.num_cores, info.num_subcores, info.num_lanes
    assert B == ncores * nsub and Cc % LN == 0
    mesh = plsc.VectorSubcoreMesh(core_axis_name="c", subcore_axis_name="s")

    @functools.partial(
        pl.kernel, mesh=mesh,
        out_type=jax.ShapeDtypeStruct((B, NQ, Cc), jnp.float32),
        scratch_types=[
            pltpu.VMEM((L, Cc), jnp.float32),
            pltpu.VMEM((NQ, Cc), jnp.float32),
        ],
    )
    def run(text_hbm, out_hbm, tin, tout):
        wid = lax.axis_index("s") * ncores + lax.axis_index("c")
        pltpu.sync_copy(text_hbm.at[wid], tin)

        def chunk(ci, carry):
            c0 = ci * LN
            vs = [tin[l, pl.ds(c0, LN)] for l in range(L)]
            for t in range(NQ):
                m = vs[0]
                for l in range(1, L):
                    m = jnp.maximum(m, vs[l])
                todo = jnp.full((LN,), 1.0, jnp.float32)
                for l in range(L):
                    hit = jnp.where(vs[l] == m, todo, 0.0)
                    vs[l] = jnp.where(hit > 0.5, NEG, vs[l])
                    todo = todo - hit
                tout[t, pl.ds(c0, LN)] = m
            return carry

        lax.fori_loop(0, Cc // LN, chunk, 0)
        pltpu.sync_copy(tout, out_hbm.at[wid])

    return run(text_feat)


def _main_body(NB, L, HW, W,
               text_ref, img_ref, posF_ref, vec_ref,
               Wq_ref, bq_ref, Wk_ref, bk_ref, M_ref,
               fW1_ref, fb1_ref, fW2_ref, fb2_ref,
               n2g_ref, n2b_ref, n3g_ref, n3b_ref, png_ref, pnb_ref,
               mW1_ref, mb1_ref, mW2_ref, mb2_ref, mW3_ref, mb3_ref,
               x_ref, pts_ref, g_ref, attn_ref):
    sel = text_ref[...]
    q1 = vec_ref[0:1, :]
    cbias = vec_ref[1:2, :]
    qh = _nn(sel.reshape(NB * NQ, C) + q1,
             Wq_ref[...]) + bq_ref[...]

    imgT = [jnp.transpose(img_ref[i]) for i in range(NB)]
    kin = jnp.concatenate([t + posF_ref[...] for t in imgT], axis=0)
    kh = _nn(kin, Wk_ref[...]) + bk_ref[...]
    ss = []
    for i in range(NB):
        qh_i = qh[i * NQ:(i + 1) * NQ, :]
        kh_i = kh[i * HW:(i + 1) * HW, :]
        ss.extend(_nt(qh_i[:, h * DH:(h + 1) * DH],
                      kh_i[:, h * DH:(h + 1) * DH]) for h in range(NH))
    s = jnp.concatenate(ss, axis=0)
    s = s / math.sqrt(DH)
    p = jax.nn.softmax(s, axis=-1)

    am = jnp.mean(p.reshape(NB, NH, NQ, HW), axis=1)
    g = jnp.max(am, axis=1, keepdims=True)
    g_ref[...] = g
    attn_ref[...] = am

    coli = jax.lax.broadcasted_iota(jnp.int32, (NB, HW), 1)
    cur = g.reshape(NB, HW)
    hots = []
    xs = []
    ys = []
    for _ in range(NQ):
        m = jnp.max(cur, axis=1, keepdims=True)
        idx = jnp.min(jnp.where(cur == m, coli, HW), axis=1, keepdims=True)
        hit = coli == idx
        hots.append(hit.astype(jnp.float32)[:, None, :])
        cur = jnp.where(hit, NEG, cur)
        xs.append((((idx % W).astype(jnp.float32) + 0.5) / W)[:, None, :])
        ys.append((((idx // W).astype(jnp.float32) + 0.5)
                   / (HW // W))[:, None, :])
    pts_ref[...] = jnp.concatenate(
        [jnp.concatenate(xs, axis=1), jnp.concatenate(ys, axis=1)], axis=2)

    oh = jnp.concatenate(hots, axis=1)
    zpad = jnp.zeros((6, HW), jnp.float32)
    ctxs = [_nn(jnp.concatenate(
                [p[i * NH * NQ:(i + 1) * NH * NQ], oh[i], zpad], axis=0),
                imgT[i]) for i in range(NB)]

    ca = cbias
    for h in range(NH):
        ch = jnp.concatenate([c[h * NQ:(h + 1) * NQ, :] for c in ctxs],
                             axis=0)
        ca = ca + _nn(ch, M_ref[h])
    q2 = _ln_rows(q1 + ca, n2g_ref[...], n2b_ref[...])
    ffn = _nn(jnp.maximum(_nn(q2, fW1_ref[...]) + fb1_ref[...], 0.0),
              fW2_ref[...]) + fb2_ref[...]
    q3 = _ln_rows(q2 + ffn, n3g_ref[...], n3b_ref[...])
    q4 = _ln_rows(q3, png_ref[...], pnb_ref[...])

    pos_feat = jnp.concatenate(
        [c[NH * NQ:NH * NQ + NQ, :] for c in ctxs], axis=0)
    x = jnp.concatenate([q4, pos_feat], axis=1)
    x = jnp.maximum(_nn(x, mW1_ref[...]) + mb1_ref[...], 0.0)
    x = jnp.maximum(_nn(x, mW2_ref[...]) + mb2_ref[...], 0.0)
    x = _nn(x, mW3_ref[...]) + mb3_ref[...]
    x_ref[...] = x.reshape(NB, NQ, C)


def kernel(text_feat, text_mask, img_feat, params):
    del text_mask
    B, L, _ = text_feat.shape
    _, _, H, W = img_feat.shape
    HW = H * W
    img = img_feat.reshape(B, C, HW)
    p = params
    r = lambda v: v.reshape(1, -1)

    sel = _sc_text_topk(text_feat)

    posF, M_all, vec = pl.pallas_call(
        functools.partial(_precompute_body, H, W),
        out_shape=[
            jax.ShapeDtypeStruct((HW, C), jnp.float32),
            jax.ShapeDtypeStruct((NH, C, C), jnp.float32),
            jax.ShapeDtypeStruct((8, C), jnp.float32),
        ],
    )(p['sa_Wo'], r(p['sa_bo']), r(p['sa_bv']), r(p['n1_g']), r(p['n1_b']),
      p['ca_Wv'], p['ca_Wo'], r(p['ca_bv']), r(p['ca_bo']))

    NB = 4
    full = lambda shape: pl.BlockSpec(shape, lambda b: (0,) * len(shape))
    perb = lambda shape: pl.BlockSpec((NB,) + shape,
                                      lambda b: (b,) + (0,) * len(shape))
    x, pts, g, am = pl.pallas_call(
        functools.partial(_main_body, NB, L, HW, W),
        grid=(B // NB,),
        in_specs=[
            perb((NQ, C)), perb((C, HW)), full((HW, C)), full((8, C)),
            full((C, C)), full((1, C)), full((C, C)), full((1, C)),
            full((NH, C, C)),
            full((C, FF)), full((1, FF)), full((FF, C)), full((1, C)),
            full((1, C)), full((1, C)), full((1, C)), full((1, C)),
            full((1, C)), full((1, C)),
            full((2 * C, C)), full((1, C)), full((C, C)), full((1, C)),
            full((C, C)), full((1, C)),
        ],
        out_specs=[perb((NQ, C)), perb((NQ, 2)), perb((1, HW)),
                   perb((NQ, HW))],
        out_shape=[
            jax.ShapeDtypeStruct((B, NQ, C), jnp.float32),
            jax.ShapeDtypeStruct((B, NQ, 2), jnp.float32),
            jax.ShapeDtypeStruct((B, 1, HW), jnp.float32),
            jax.ShapeDtypeStruct((B, NQ, HW), jnp.float32),
        ],
    )(sel, img, posF, vec,
      p['ca_Wq'], r(p['ca_bq']), p['ca_Wk'], r(p['ca_bk']), M_all,
      p['ffn_W1'], r(p['ffn_b1']), p['ffn_W2'], r(p['ffn_b2']),
      r(p['n2_g']), r(p['n2_b']), r(p['n3_g']), r(p['n3_b']),
      r(p['pn_g']), r(p['pn_b']),
      p['mlp_W1'], r(p['mlp_b1']), p['mlp_W2'], r(p['mlp_b2']),
      p['mlp_W3'], r(p['mlp_b3']))

    return (x, pts, g.reshape(B, H, W), am.reshape(B, NQ, H, W))


# speedup vs baseline: 1.1061x; 1.1061x over previous
import functools
import math

import jax
import jax.numpy as jnp
from jax import lax
from jax.experimental import pallas as pl
from jax.experimental.pallas import tpu as pltpu
from jax.experimental.pallas import tpu_sc as plsc

C = 256
NQ = 10
NH = 8
DH = C // NH
FF = 512
NEG = float("-inf")


def _ln_rows(x, g, b):
    m = jnp.mean(x, axis=-1, keepdims=True)
    v = jnp.mean((x - m) ** 2, axis=-1, keepdims=True)
    return (x - m) / jnp.sqrt(v + 1e-5) * g + b


def _nn(a, b):
    return jax.lax.dot_general(a, b, (((1,), (0,)), ((), ())),
                               preferred_element_type=jnp.float32)


def _nt(a, b):
    return jax.lax.dot_general(a, b, (((1,), (1,)), ((), ())),
                               preferred_element_type=jnp.float32)


def _precompute_body(H, W, sa_Wo, sa_bo, sa_bv, n1g, n1b,
                     ca_Wv, ca_Wo, ca_bv, ca_bo,
                     posF_ref, M_ref, vec_ref):
    HW = H * W
    ci = jax.lax.broadcasted_iota(jnp.int32, (C, HW), 0)
    pi = jax.lax.broadcasted_iota(jnp.int32, (C, HW), 1)
    i = pi // W
    j = pi % W
    scale = 2.0 * math.pi
    yv = (i.astype(jnp.float32) + 1.0) / (H + 1e-6) * scale
    xv = (j.astype(jnp.float32) + 1.0) / (W + 1e-6) * scale
    k = (ci % (C // 2)) // 2
    tw = jnp.exp(k.astype(jnp.float32) * (2.0 / (C // 2)) * math.log(10000.0))
    val = jnp.where(ci < (C // 2), yv, xv) / tw
    posF_ref[...] = jnp.where(ci % 2 == 0, jnp.sin(val), jnp.cos(val))
    for h in range(NH):
        M_ref[h] = _nn(ca_Wv[:, h * DH:(h + 1) * DH],
                       ca_Wo[h * DH:(h + 1) * DH, :])
    c0 = _nn(sa_bv[...], sa_Wo[...]) + sa_bo[...]
    q1 = _ln_rows(c0, n1g[...], n1b[...])
    cb = _nn(ca_bv[...], ca_Wo[...]) + ca_bo[...]
    vec_ref[...] = jnp.concatenate(
        [q1, cb, jnp.zeros((6, C), jnp.float32)], axis=0)


def _sc_text_topk(text_feat):
    B, L, Cc = text_feat.shape
    info = plsc.get_sparse_core_info()
    ncores, nsub, LN = info.num_cores, info.num_subcores, info.num_lanes
    assert B == ncores * nsub and Cc % LN == 0
    mesh = plsc.VectorSubcoreMesh(core_axis_name="c", subcore_axis_name="s")

    @functools.partial(
        pl.kernel, mesh=mesh,
        out_type=jax.ShapeDtypeStruct((B, NQ, Cc), jnp.float32),
        scratch_types=[
            pltpu.VMEM((L, Cc), jnp.float32),
            pltpu.VMEM((NQ, Cc), jnp.float32),
        ],
    )
    def run(text_hbm, out_hbm, tin, tout):
        wid = lax.axis_index("s") * ncores + lax.axis_index("c")
        pltpu.sync_copy(text_hbm.at[wid], tin)

        def chunk(ci, carry):
            c0 = ci * LN
            vs = [tin[l, pl.ds(c0, LN)] for l in range(L)]
            for t in range(NQ):
                m = vs[0]
                for l in range(1, L):
                    m = jnp.maximum(m, vs[l])
                todo = jnp.full((LN,), 1.0, jnp.float32)
                for l in range(L):
                    hit = jnp.where(vs[l] == m, todo, 0.0)
                    vs[l] = jnp.where(hit > 0.5, NEG, vs[l])
                    todo = todo - hit
                tout[t, pl.ds(c0, LN)] = m
            return carry

        lax.fori_loop(0, Cc // LN, chunk, 0)
        pltpu.sync_copy(tout, out_hbm.at[wid])

    return run(text_feat)


def _main_body(NB, L, HW, W,
               text_ref, img_ref, posF_ref, vec_ref,
               Wq_ref, bq_ref, WkT_ref, bk_ref, M_ref,
               fW1_ref, fb1_ref, fW2_ref, fb2_ref,
               n2g_ref, n2b_ref, n3g_ref, n3b_ref, png_ref, pnb_ref,
               mW1_ref, mb1_ref, mW2_ref, mb2_ref, mW3_ref, mb3_ref,
               x_ref, pts_ref, g_ref, attn_ref):
    sel = text_ref[...]
    q1 = vec_ref[0:1, :]
    cbias = vec_ref[1:2, :]
    qh = _nn(sel.reshape(NB * NQ, C) + q1,
             Wq_ref[...]) + bq_ref[...]

    khts = [_nn(WkT_ref[...], img_ref[i] + posF_ref[...]) + bk_ref[...]
            for i in range(NB)]
    ss = []
    for i in range(NB):
        qh_i = qh[i * NQ:(i + 1) * NQ, :]
        ss.extend(_nn(qh_i[:, h * DH:(h + 1) * DH],
                      khts[i][h * DH:(h + 1) * DH, :]) for h in range(NH))
    s = jnp.concatenate(ss, axis=0)
    s = s / math.sqrt(DH)
    p = jax.nn.softmax(s, axis=-1)

    am = jnp.mean(p.reshape(NB, NH, NQ, HW), axis=1)
    g = jnp.max(am, axis=1, keepdims=True)
    g_ref[...] = g
    attn_ref[...] = am

    coli = jax.lax.broadcasted_iota(jnp.int32, (NB, HW), 1)
    cur = g.reshape(NB, HW)
    hots = []
    xs = []
    ys = []
    for _ in range(NQ):
        m = jnp.max(cur, axis=1, keepdims=True)
        idx = jnp.min(jnp.where(cur == m, coli, HW), axis=1, keepdims=True)
        hit = coli == idx
        hots.append(hit.astype(jnp.float32)[:, None, :])
        cur = jnp.where(hit, NEG, cur)
        xs.append((((idx % W).astype(jnp.float32) + 0.5) / W)[:, None, :])
        ys.append((((idx // W).astype(jnp.float32) + 0.5)
                   / (HW // W))[:, None, :])
    pts_ref[...] = jnp.concatenate(
        [jnp.concatenate(xs, axis=1), jnp.concatenate(ys, axis=1)], axis=2)

    oh = jnp.concatenate(hots, axis=1)
    zpad = jnp.zeros((6, HW), jnp.float32)
    ctxs = [_nt(jnp.concatenate(
                [p[i * NH * NQ:(i + 1) * NH * NQ], oh[i], zpad], axis=0),
                img_ref[i]) for i in range(NB)]

    ca = cbias
    for h in range(NH):
        ch = jnp.concatenate([c[h * NQ:(h + 1) * NQ, :] for c in ctxs],
                             axis=0)
        ca = ca + _nn(ch, M_ref[h])
    q2 = _ln_rows(q1 + ca, n2g_ref[...], n2b_ref[...])
    ffn = _nn(jnp.maximum(_nn(q2, fW1_ref[...]) + fb1_ref[...], 0.0),
              fW2_ref[...]) + fb2_ref[...]
    q3 = _ln_rows(q2 + ffn, n3g_ref[...], n3b_ref[...])
    q4 = _ln_rows(q3, png_ref[...], pnb_ref[...])

    pos_feat = jnp.concatenate(
        [c[NH * NQ:NH * NQ + NQ, :] for c in ctxs], axis=0)
    x = jnp.concatenate([q4, pos_feat], axis=1)
    x = jnp.maximum(_nn(x, mW1_ref[...]) + mb1_ref[...], 0.0)
    x = jnp.maximum(_nn(x, mW2_ref[...]) + mb2_ref[...], 0.0)
    x = _nn(x, mW3_ref[...]) + mb3_ref[...]
    x_ref[...] = x.reshape(NB, NQ, C)


def kernel(text_feat, text_mask, img_feat, params):
    del text_mask
    B, L, _ = text_feat.shape
    _, _, H, W = img_feat.shape
    HW = H * W
    img = img_feat.reshape(B, C, HW)
    p = params
    r = lambda v: v.reshape(1, -1)

    sel = _sc_text_topk(text_feat)

    posF, M_all, vec = pl.pallas_call(
        functools.partial(_precompute_body, H, W),
        out_shape=[
            jax.ShapeDtypeStruct((C, HW), jnp.float32),
            jax.ShapeDtypeStruct((NH, C, C), jnp.float32),
            jax.ShapeDtypeStruct((8, C), jnp.float32),
        ],
    )(p['sa_Wo'], r(p['sa_bo']), r(p['sa_bv']), r(p['n1_g']), r(p['n1_b']),
      p['ca_Wv'], p['ca_Wo'], r(p['ca_bv']), r(p['ca_bo']))

    NB = 4
    full = lambda shape: pl.BlockSpec(shape, lambda b: (0,) * len(shape))
    perb = lambda shape: pl.BlockSpec((NB,) + shape,
                                      lambda b: (b,) + (0,) * len(shape))
    x, pts, g, am = pl.pallas_call(
        functools.partial(_main_body, NB, L, HW, W),
        grid=(B // NB,),
        in_specs=[
            perb((NQ, C)), perb((C, HW)), full((C, HW)), full((8, C)),
            full((C, C)), full((1, C)), full((C, C)), full((C, 1)),
            full((NH, C, C)),
            full((C, FF)), full((1, FF)), full((FF, C)), full((1, C)),
            full((1, C)), full((1, C)), full((1, C)), full((1, C)),
            full((1, C)), full((1, C)),
            full((2 * C, C)), full((1, C)), full((C, C)), full((1, C)),
            full((C, C)), full((1, C)),
        ],
        out_specs=[perb((NQ, C)), perb((NQ, 2)), perb((1, HW)),
                   perb((NQ, HW))],
        out_shape=[
            jax.ShapeDtypeStruct((B, NQ, C), jnp.float32),
            jax.ShapeDtypeStruct((B, NQ, 2), jnp.float32),
            jax.ShapeDtypeStruct((B, 1, HW), jnp.float32),
            jax.ShapeDtypeStruct((B, NQ, HW), jnp.float32),
        ],
    )(sel, img, posF, vec,
      p['ca_Wq'], r(p['ca_bq']), p['ca_Wk'].T, p['ca_bk'].reshape(C, 1),
      M_all,
      p['ffn_W1'], r(p['ffn_b1']), p['ffn_W2'], r(p['ffn_b2']),
      r(p['n2_g']), r(p['n2_b']), r(p['n3_g']), r(p['n3_b']),
      r(p['pn_g']), r(p['pn_b']),
      p['mlp_W1'], r(p['mlp_b1']), p['mlp_W2'], r(p['mlp_b2']),
      p['mlp_W3'], r(p['mlp_b3']))

    return (x, pts, g.reshape(B, H, W), am.reshape(B, NQ, H, W))


# speedup vs baseline: 1.5140x; 1.3687x over previous
import functools
import math

import jax
import jax.numpy as jnp
from jax import lax
from jax.experimental import pallas as pl
from jax.experimental.pallas import tpu as pltpu
from jax.experimental.pallas import tpu_sc as plsc

C = 256
NQ = 10
NH = 8
DH = C // NH
FF = 512
NEG = float("-inf")


def _ln_rows(x, g, b):
    m = jnp.mean(x, axis=-1, keepdims=True)
    v = jnp.mean((x - m) ** 2, axis=-1, keepdims=True)
    return (x - m) / jnp.sqrt(v + 1e-5) * g + b


def _nn(a, b):
    return jax.lax.dot_general(a, b, (((1,), (0,)), ((), ())),
                               preferred_element_type=jnp.float32)


def _nt(a, b):
    return jax.lax.dot_general(a, b, (((1,), (1,)), ((), ())),
                               preferred_element_type=jnp.float32)


def _precompute_body(H, W, sa_Wo, sa_bo, sa_bv, n1g, n1b,
                     ca_Wv, ca_Wo, ca_bv, ca_bo,
                     posF_ref, M_ref, vec_ref):
    HW = H * W
    ci = jax.lax.broadcasted_iota(jnp.int32, (HW, C), 1)
    pi = jax.lax.broadcasted_iota(jnp.int32, (HW, C), 0)
    i = pi // W
    j = pi % W
    scale = 2.0 * math.pi
    yv = (i.astype(jnp.float32) + 1.0) / (H + 1e-6) * scale
    xv = (j.astype(jnp.float32) + 1.0) / (W + 1e-6) * scale
    k = (ci % (C // 2)) // 2
    tw = jnp.exp(k.astype(jnp.float32) * (2.0 / (C // 2)) * math.log(10000.0))
    val = jnp.where(ci < (C // 2), yv, xv) / tw
    posF_ref[...] = jnp.where(ci % 2 == 0, jnp.sin(val), jnp.cos(val))
    for h in range(NH):
        M_ref[h] = _nn(ca_Wv[:, h * DH:(h + 1) * DH],
                       ca_Wo[h * DH:(h + 1) * DH, :])
    c0 = _nn(sa_bv[...], sa_Wo[...]) + sa_bo[...]
    q1 = _ln_rows(c0, n1g[...], n1b[...])
    cb = _nn(ca_bv[...], ca_Wo[...]) + ca_bo[...]
    vec_ref[...] = jnp.concatenate(
        [q1, cb, jnp.zeros((6, C), jnp.float32)], axis=0)


def _sc_text_topk(text_feat):
    B, L, Cc = text_feat.shape
    info = plsc.get_sparse_core_info()
    ncores, nsub, LN = info.num_cores, info.num_subcores, info.num_lanes
    assert B == ncores * nsub and Cc % LN == 0
    mesh = plsc.VectorSubcoreMesh(core_axis_name="c", subcore_axis_name="s")

    @functools.partial(
        pl.kernel, mesh=mesh,
        out_type=jax.ShapeDtypeStruct((B, NQ, Cc), jnp.float32),
        scratch_types=[
            pltpu.VMEM((L, Cc), jnp.float32),
            pltpu.VMEM((NQ, Cc), jnp.float32),
        ],
    )
    def run(text_hbm, out_hbm, tin, tout):
        wid = lax.axis_index("s") * ncores + lax.axis_index("c")
        pltpu.sync_copy(text_hbm.at[wid], tin)

        def chunk(ci, carry):
            c0 = ci * LN
            vs = [tin[l, pl.ds(c0, LN)] for l in range(L)]
            for t in range(NQ):
                m = vs[0]
                for l in range(1, L):
                    m = jnp.maximum(m, vs[l])
                todo = jnp.full((LN,), 1.0, jnp.float32)
                for l in range(L):
                    hit = jnp.where(vs[l] == m, todo, 0.0)
                    vs[l] = jnp.where(hit > 0.5, NEG, vs[l])
                    todo = todo - hit
                tout[t, pl.ds(c0, LN)] = m
            return carry

        lax.fori_loop(0, Cc // LN, chunk, 0)
        pltpu.sync_copy(tout, out_hbm.at[wid])

    return run(text_feat)


def _main_body(NB, L, HW, W,
               text_ref, img_ref, posF_ref, vec_ref,
               Wq_ref, bq_ref, Wk_ref, bk_ref, M_ref,
               fW1_ref, fb1_ref, fW2_ref, fb2_ref,
               n2g_ref, n2b_ref, n3g_ref, n3b_ref, png_ref, pnb_ref,
               mW1_ref, mb1_ref, mW2_ref, mb2_ref, mW3_ref, mb3_ref,
               x_ref, pts_ref, g_ref, attn_ref):
    sel = text_ref[...]
    q1 = vec_ref[0:1, :]
    cbias = vec_ref[1:2, :]
    qh = _nn(sel.reshape(NB * NQ, C) + q1,
             Wq_ref[...]) + bq_ref[...]

    kin = (img_ref[...] + posF_ref[...][None]).reshape(NB * HW, C)
    kh = _nn(kin, Wk_ref[...]) + bk_ref[...]
    ss = []
    for i in range(NB):
        qh_i = qh[i * NQ:(i + 1) * NQ, :]
        kh_i = kh[i * HW:(i + 1) * HW, :]
        ss.extend(_nt(qh_i[:, h * DH:(h + 1) * DH],
                      kh_i[:, h * DH:(h + 1) * DH]) for h in range(NH))
    s = jnp.concatenate(ss, axis=0)
    s = s / math.sqrt(DH)
    p = jax.nn.softmax(s, axis=-1)

    am = jnp.mean(p.reshape(NB, NH, NQ, HW), axis=1)
    g = jnp.max(am, axis=1, keepdims=True)
    g_ref[...] = g
    attn_ref[...] = am

    coli = jax.lax.broadcasted_iota(jnp.int32, (NB, HW), 1)
    cur = g.reshape(NB, HW)
    hots = []
    xs = []
    ys = []
    for _ in range(NQ):
        m = jnp.max(cur, axis=1, keepdims=True)
        idx = jnp.min(jnp.where(cur == m, coli, HW), axis=1, keepdims=True)
        hit = coli == idx
        hots.append(hit.astype(jnp.float32)[:, None, :])
        cur = jnp.where(hit, NEG, cur)
        xs.append((((idx % W).astype(jnp.float32) + 0.5) / W)[:, None, :])
        ys.append((((idx // W).astype(jnp.float32) + 0.5)
                   / (HW // W))[:, None, :])
    pts_ref[...] = jnp.concatenate(
        [jnp.concatenate(xs, axis=1), jnp.concatenate(ys, axis=1)], axis=2)

    oh = jnp.concatenate(hots, axis=1)
    zpad = jnp.zeros((6, HW), jnp.float32)
    ctxs = [_nn(jnp.concatenate(
                [p[i * NH * NQ:(i + 1) * NH * NQ], oh[i], zpad], axis=0),
                img_ref[i]) for i in range(NB)]

    ca = cbias
    for h in range(NH):
        ch = jnp.concatenate([c[h * NQ:(h + 1) * NQ, :] for c in ctxs],
                             axis=0)
        ca = ca + _nn(ch, M_ref[h])
    q2 = _ln_rows(q1 + ca, n2g_ref[...], n2b_ref[...])
    ffn = _nn(jnp.maximum(_nn(q2, fW1_ref[...]) + fb1_ref[...], 0.0),
              fW2_ref[...]) + fb2_ref[...]
    q3 = _ln_rows(q2 + ffn, n3g_ref[...], n3b_ref[...])
    q4 = _ln_rows(q3, png_ref[...], pnb_ref[...])

    pos_feat = jnp.concatenate(
        [c[NH * NQ:NH * NQ + NQ, :] for c in ctxs], axis=0)
    x = jnp.concatenate([q4, pos_feat], axis=1)
    x = jnp.maximum(_nn(x, mW1_ref[...]) + mb1_ref[...], 0.0)
    x = jnp.maximum(_nn(x, mW2_ref[...]) + mb2_ref[...], 0.0)
    x = _nn(x, mW3_ref[...]) + mb3_ref[...]
    x_ref[...] = x.reshape(NB, NQ, C)


def kernel(text_feat, text_mask, img_feat, params):
    del text_mask
    B, L, _ = text_feat.shape
    _, _, H, W = img_feat.shape
    HW = H * W
    img = img_feat.reshape(B, C, HW).transpose(0, 2, 1)
    p = params
    r = lambda v: v.reshape(1, -1)

    sel = _sc_text_topk(text_feat)

    posF, M_all, vec = pl.pallas_call(
        functools.partial(_precompute_body, H, W),
        out_shape=[
            jax.ShapeDtypeStruct((HW, C), jnp.float32),
            jax.ShapeDtypeStruct((NH, C, C), jnp.float32),
            jax.ShapeDtypeStruct((8, C), jnp.float32),
        ],
    )(p['sa_Wo'], r(p['sa_bo']), r(p['sa_bv']), r(p['n1_g']), r(p['n1_b']),
      p['ca_Wv'], p['ca_Wo'], r(p['ca_bv']), r(p['ca_bo']))

    NB = 8
    full = lambda shape: pl.BlockSpec(shape, lambda b: (0,) * len(shape))
    perb = lambda shape: pl.BlockSpec((NB,) + shape,
                                      lambda b: (b,) + (0,) * len(shape))
    x, pts, g, am = pl.pallas_call(
        functools.partial(_main_body, NB, L, HW, W),
        grid=(B // NB,),
        in_specs=[
            perb((NQ, C)), perb((HW, C)), full((HW, C)), full((8, C)),
            full((C, C)), full((1, C)), full((C, C)), full((1, C)),
            full((NH, C, C)),
            full((C, FF)), full((1, FF)), full((FF, C)), full((1, C)),
            full((1, C)), full((1, C)), full((1, C)), full((1, C)),
            full((1, C)), full((1, C)),
            full((2 * C, C)), full((1, C)), full((C, C)), full((1, C)),
            full((C, C)), full((1, C)),
        ],
        out_specs=[perb((NQ, C)), perb((NQ, 2)), perb((1, HW)),
                   perb((NQ, HW))],
        out_shape=[
            jax.ShapeDtypeStruct((B, NQ, C), jnp.float32),
            jax.ShapeDtypeStruct((B, NQ, 2), jnp.float32),
            jax.ShapeDtypeStruct((B, 1, HW), jnp.float32),
            jax.ShapeDtypeStruct((B, NQ, HW), jnp.float32),
        ],
    )(sel, img, posF, vec,
      p['ca_Wq'], r(p['ca_bq']), p['ca_Wk'], r(p['ca_bk']), M_all,
      p['ffn_W1'], r(p['ffn_b1']), p['ffn_W2'], r(p['ffn_b2']),
      r(p['n2_g']), r(p['n2_b']), r(p['n3_g']), r(p['n3_b']),
      r(p['pn_g']), r(p['pn_b']),
      p['mlp_W1'], r(p['mlp_b1']), p['mlp_W2'], r(p['mlp_b2']),
      p['mlp_W3'], r(p['mlp_b3']))

    return (x, pts, g.reshape(B, H, W), am.reshape(B, NQ, H, W))


# speedup vs baseline: 1.7691x; 1.1685x over previous
import functools
import math

import jax
import jax.numpy as jnp
from jax import lax
from jax.experimental import pallas as pl
from jax.experimental.pallas import tpu as pltpu
from jax.experimental.pallas import tpu_sc as plsc

C = 256
NQ = 10
NH = 8
DH = C // NH
FF = 512
NEG = float("-inf")


def _ln_rows(x, g, b):
    m = jnp.mean(x, axis=-1, keepdims=True)
    v = jnp.mean((x - m) ** 2, axis=-1, keepdims=True)
    return (x - m) / jnp.sqrt(v + 1e-5) * g + b


def _nn(a, b):
    return jax.lax.dot_general(a, b, (((1,), (0,)), ((), ())),
                               preferred_element_type=jnp.float32)


def _nt(a, b):
    return jax.lax.dot_general(a, b, (((1,), (1,)), ((), ())),
                               preferred_element_type=jnp.float32)


def _precompute_body(H, W, sa_Wo, sa_bo, sa_bv, n1g, n1b,
                     ca_Wv, ca_Wo, ca_bv, ca_bo,
                     posF_ref, M_ref, vec_ref):
    HW = H * W
    ci = jax.lax.broadcasted_iota(jnp.int32, (HW, C), 1)
    pi = jax.lax.broadcasted_iota(jnp.int32, (HW, C), 0)
    i = pi // W
    j = pi % W
    scale = 2.0 * math.pi
    yv = (i.astype(jnp.float32) + 1.0) / (H + 1e-6) * scale
    xv = (j.astype(jnp.float32) + 1.0) / (W + 1e-6) * scale
    k = (ci % (C // 2)) // 2
    tw = jnp.exp(k.astype(jnp.float32) * (2.0 / (C // 2)) * math.log(10000.0))
    val = jnp.where(ci < (C // 2), yv, xv) / tw
    posF_ref[...] = jnp.where(ci % 2 == 0, jnp.sin(val), jnp.cos(val))
    for h in range(NH):
        M_ref[h] = _nn(ca_Wv[:, h * DH:(h + 1) * DH],
                       ca_Wo[h * DH:(h + 1) * DH, :])
    c0 = _nn(sa_bv[...], sa_Wo[...]) + sa_bo[...]
    q1 = _ln_rows(c0, n1g[...], n1b[...])
    cb = _nn(ca_bv[...], ca_Wo[...]) + ca_bo[...]
    vec_ref[...] = jnp.concatenate(
        [q1, cb, jnp.zeros((6, C), jnp.float32)], axis=0)


def _sc_text_topk(text_feat):
    B, L, Cc = text_feat.shape
    info = plsc.get_sparse_core_info()
    ncores, nsub, LN = info.num_cores, info.num_subcores, info.num_lanes
    assert B == ncores * nsub and Cc % LN == 0
    mesh = plsc.VectorSubcoreMesh(core_axis_name="c", subcore_axis_name="s")

    @functools.partial(
        pl.kernel, mesh=mesh,
        out_type=jax.ShapeDtypeStruct((B, NQ, Cc), jnp.float32),
        scratch_types=[
            pltpu.VMEM((L, Cc), jnp.float32),
            pltpu.VMEM((NQ, Cc), jnp.float32),
        ],
    )
    def run(text_hbm, out_hbm, tin, tout):
        wid = lax.axis_index("s") * ncores + lax.axis_index("c")
        pltpu.sync_copy(text_hbm.at[wid], tin)

        def chunk(ci, carry):
            c0 = ci * LN
            vs = [tin[l, pl.ds(c0, LN)] for l in range(L)]
            for t in range(NQ):
                m = vs[0]
                for l in range(1, L):
                    m = jnp.maximum(m, vs[l])
                todo = jnp.full((LN,), 1.0, jnp.float32)
                for l in range(L):
                    hit = jnp.where(vs[l] == m, todo, 0.0)
                    vs[l] = jnp.where(hit > 0.5, NEG, vs[l])
                    todo = todo - hit
                tout[t, pl.ds(c0, LN)] = m
            return carry

        lax.fori_loop(0, Cc // LN, chunk, 0)
        pltpu.sync_copy(tout, out_hbm.at[wid])

    return run(text_feat)


def _main_body(NB, L, HW, W,
               text_ref, img_ref, posF_ref, vec_ref,
               Wq_ref, bq_ref, Wk_ref, bk_ref, M_ref,
               fW1_ref, fb1_ref, fW2_ref, fb2_ref,
               n2g_ref, n2b_ref, n3g_ref, n3b_ref, png_ref, pnb_ref,
               mW1_ref, mb1_ref, mW2_ref, mb2_ref, mW3_ref, mb3_ref,
               x_ref, pts_ref, g_ref, attn_ref):
    sel = text_ref[...]
    q1 = vec_ref[0:1, :]
    cbias = vec_ref[1:2, :]
    qh = _nn(sel.reshape(NB * NQ, C) + q1,
             Wq_ref[...]) + bq_ref[...]

    kin = (img_ref[...] + posF_ref[...][None]).reshape(NB * HW, C)
    kh = _nn(kin, Wk_ref[...]) + bk_ref[...]
    colh = jax.lax.broadcasted_iota(jnp.int32, (1, C), 1) // DH
    ss = []
    for i in range(NB):
        qh_i = qh[i * NQ:(i + 1) * NQ, :]
        qblk = jnp.concatenate(
            [jnp.where(colh == h, qh_i, 0.0) for h in range(NH)], axis=0)
        ss.append(_nt(qblk, kh[i * HW:(i + 1) * HW, :]))
    s = jnp.concatenate(ss, axis=0)
    s = s / math.sqrt(DH)
    p = jax.nn.softmax(s, axis=-1)

    am = jnp.mean(p.reshape(NB, NH, NQ, HW), axis=1)
    g = jnp.max(am, axis=1, keepdims=True)
    g_ref[...] = g
    attn_ref[...] = am

    coli = jax.lax.broadcasted_iota(jnp.int32, (NB, HW), 1)
    cur = g.reshape(NB, HW)
    hots = []
    xs = []
    ys = []
    for _ in range(NQ):
        m = jnp.max(cur, axis=1, keepdims=True)
        idx = jnp.min(jnp.where(cur == m, coli, HW), axis=1, keepdims=True)
        hit = coli == idx
        hots.append(hit.astype(jnp.float32)[:, None, :])
        cur = jnp.where(hit, NEG, cur)
        xs.append((((idx % W).astype(jnp.float32) + 0.5) / W)[:, None, :])
        ys.append((((idx // W).astype(jnp.float32) + 0.5)
                   / (HW // W))[:, None, :])
    pts_ref[...] = jnp.concatenate(
        [jnp.concatenate(xs, axis=1), jnp.concatenate(ys, axis=1)], axis=2)

    oh = jnp.concatenate(hots, axis=1)
    zpad = jnp.zeros((6, HW), jnp.float32)
    ctxs = [_nn(jnp.concatenate(
                [p[i * NH * NQ:(i + 1) * NH * NQ], oh[i], zpad], axis=0),
                img_ref[i]) for i in range(NB)]

    ca = cbias
    for h in range(NH):
        ch = jnp.concatenate([c[h * NQ:(h + 1) * NQ, :] for c in ctxs],
                             axis=0)
        ca = ca + _nn(ch, M_ref[h])
    q2 = _ln_rows(q1 + ca, n2g_ref[...], n2b_ref[...])
    ffn = _nn(jnp.maximum(_nn(q2, fW1_ref[...]) + fb1_ref[...], 0.0),
              fW2_ref[...]) + fb2_ref[...]
    q3 = _ln_rows(q2 + ffn, n3g_ref[...], n3b_ref[...])
    q4 = _ln_rows(q3, png_ref[...], pnb_ref[...])

    pos_feat = jnp.concatenate(
        [c[NH * NQ:NH * NQ + NQ, :] for c in ctxs], axis=0)
    x = jnp.concatenate([q4, pos_feat], axis=1)
    x = jnp.maximum(_nn(x, mW1_ref[...]) + mb1_ref[...], 0.0)
    x = jnp.maximum(_nn(x, mW2_ref[...]) + mb2_ref[...], 0.0)
    x = _nn(x, mW3_ref[...]) + mb3_ref[...]
    x_ref[...] = x.reshape(NB, NQ, C)


def kernel(text_feat, text_mask, img_feat, params):
    del text_mask
    B, L, _ = text_feat.shape
    _, _, H, W = img_feat.shape
    HW = H * W
    img = img_feat.reshape(B, C, HW).transpose(0, 2, 1)
    p = params
    r = lambda v: v.reshape(1, -1)

    sel = _sc_text_topk(text_feat)

    posF, M_all, vec = pl.pallas_call(
        functools.partial(_precompute_body, H, W),
        out_shape=[
            jax.ShapeDtypeStruct((HW, C), jnp.float32),
            jax.ShapeDtypeStruct((NH, C, C), jnp.float32),
            jax.ShapeDtypeStruct((8, C), jnp.float32),
        ],
    )(p['sa_Wo'], r(p['sa_bo']), r(p['sa_bv']), r(p['n1_g']), r(p['n1_b']),
      p['ca_Wv'], p['ca_Wo'], r(p['ca_bv']), r(p['ca_bo']))

    NB = 8
    full = lambda shape: pl.BlockSpec(shape, lambda b: (0,) * len(shape))
    perb = lambda shape: pl.BlockSpec((NB,) + shape,
                                      lambda b: (b,) + (0,) * len(shape))
    x, pts, g, am = pl.pallas_call(
        functools.partial(_main_body, NB, L, HW, W),
        grid=(B // NB,),
        in_specs=[
            perb((NQ, C)), perb((HW, C)), full((HW, C)), full((8, C)),
            full((C, C)), full((1, C)), full((C, C)), full((1, C)),
            full((NH, C, C)),
            full((C, FF)), full((1, FF)), full((FF, C)), full((1, C)),
            full((1, C)), full((1, C)), full((1, C)), full((1, C)),
            full((1, C)), full((1, C)),
            full((2 * C, C)), full((1, C)), full((C, C)), full((1, C)),
            full((C, C)), full((1, C)),
        ],
        out_specs=[perb((NQ, C)), perb((NQ, 2)), perb((1, HW)),
                   perb((NQ, HW))],
        out_shape=[
            jax.ShapeDtypeStruct((B, NQ, C), jnp.float32),
            jax.ShapeDtypeStruct((B, NQ, 2), jnp.float32),
            jax.ShapeDtypeStruct((B, 1, HW), jnp.float32),
            jax.ShapeDtypeStruct((B, NQ, HW), jnp.float32),
        ],
    )(sel, img, posF, vec,
      p['ca_Wq'], r(p['ca_bq']), p['ca_Wk'], r(p['ca_bk']), M_all,
      p['ffn_W1'], r(p['ffn_b1']), p['ffn_W2'], r(p['ffn_b2']),
      r(p['n2_g']), r(p['n2_b']), r(p['n3_g']), r(p['n3_b']),
      r(p['pn_g']), r(p['pn_b']),
      p['mlp_W1'], r(p['mlp_b1']), p['mlp_W2'], r(p['mlp_b2']),
      p['mlp_W3'], r(p['mlp_b3']))

    return (x, pts, g.reshape(B, H, W), am.reshape(B, NQ, H, W))


# speedup vs baseline: 1.9018x; 1.0750x over previous
import functools
import math

import jax
import jax.numpy as jnp
from jax import lax
from jax.experimental import pallas as pl
from jax.experimental.pallas import tpu as pltpu
from jax.experimental.pallas import tpu_sc as plsc

C = 256
NQ = 10
NH = 8
DH = C // NH
FF = 512
NEG = float("-inf")


def _ln_rows(x, g, b):
    m = jnp.mean(x, axis=-1, keepdims=True)
    v = jnp.mean((x - m) ** 2, axis=-1, keepdims=True)
    return (x - m) / jnp.sqrt(v + 1e-5) * g + b


def _nn(a, b):
    return jax.lax.dot_general(a, b, (((1,), (0,)), ((), ())),
                               preferred_element_type=jnp.float32)


def _nt(a, b):
    return jax.lax.dot_general(a, b, (((1,), (1,)), ((), ())),
                               preferred_element_type=jnp.float32)


def _precompute_body(H, W, sa_Wo, sa_bo, sa_bv, n1g, n1b,
                     ca_Wv, ca_Wo, ca_bv, ca_bo,
                     posF_ref, M_ref, vec_ref):
    HW = H * W
    HC = C // 2
    ck = jax.lax.broadcasted_iota(jnp.int32, (H, HC), 1)
    cc = jax.lax.broadcasted_iota(jnp.int32, (H, HC), 0).astype(jnp.float32)
    scale = 2.0 * math.pi
    tw = jnp.exp((ck // 2).astype(jnp.float32)
                 * (2.0 / HC) * math.log(10000.0))
    yval = ((cc + 1.0) / (H + 1e-6) * scale) / tw
    xval = ((cc + 1.0) / (W + 1e-6) * scale) / tw
    ytab = jnp.where(ck % 2 == 0, jnp.sin(yval), jnp.cos(yval))
    xtab = jnp.where(ck % 2 == 0, jnp.sin(xval), jnp.cos(xval))
    yexp = jnp.broadcast_to(ytab[:, None, :], (H, W, HC)).reshape(HW, HC)
    xexp = jnp.broadcast_to(xtab[None, :, :], (H, W, HC)).reshape(HW, HC)
    posF_ref[...] = jnp.concatenate([yexp, xexp], axis=1)
    for h in range(NH):
        M_ref[h] = _nn(ca_Wv[:, h * DH:(h + 1) * DH],
                       ca_Wo[h * DH:(h + 1) * DH, :])
    c0 = _nn(sa_bv[...], sa_Wo[...]) + sa_bo[...]
    q1 = _ln_rows(c0, n1g[...], n1b[...])
    cb = _nn(ca_bv[...], ca_Wo[...]) + ca_bo[...]
    vec_ref[...] = jnp.concatenate(
        [q1, cb, jnp.zeros((6, C), jnp.float32)], axis=0)


def _oddeven_merge(lo, n, r):
    step = r * 2
    if step < n:
        yield from _oddeven_merge(lo, n, step)
        yield from _oddeven_merge(lo + r, n, step)
        for i in range(lo + r, lo + n - r, step):
            yield (i, i + r)
    else:
        yield (lo, lo + r)


def _oddeven_sort_pairs(lo, n):
    if n > 1:
        m = n // 2
        yield from _oddeven_sort_pairs(lo, m)
        yield from _oddeven_sort_pairs(lo + m, m)
        yield from _oddeven_merge(lo, n, 1)


def _pruned_sort_pairs(n_real, n_pad):
    return [(i, j) for (i, j) in _oddeven_sort_pairs(0, n_pad)
            if i < n_real and j < n_real]


def _sc_text_topk(text_feat):
    B, L, Cc = text_feat.shape
    info = plsc.get_sparse_core_info()
    ncores, nsub, LN = info.num_cores, info.num_subcores, info.num_lanes
    assert B == ncores * nsub and Cc % LN == 0
    mesh = plsc.VectorSubcoreMesh(core_axis_name="c", subcore_axis_name="s")

    @functools.partial(
        pl.kernel, mesh=mesh,
        out_type=jax.ShapeDtypeStruct((B, NQ, Cc), jnp.float32),
        scratch_types=[
            pltpu.VMEM((L, Cc), jnp.float32),
            pltpu.VMEM((NQ, Cc), jnp.float32),
        ],
    )
    def run(text_hbm, out_hbm, tin, tout):
        wid = lax.axis_index("s") * ncores + lax.axis_index("c")
        pltpu.sync_copy(text_hbm.at[wid], tin)

        npad = 1 << (L - 1).bit_length()
        pairs = _pruned_sort_pairs(L, npad)

        def chunk(ci, carry):
            c0 = ci * LN
            vs = [tin[l, pl.ds(c0, LN)] for l in range(L)]
            for (a, b) in pairs:
                lo = jnp.minimum(vs[a], vs[b])
                hi = jnp.maximum(vs[a], vs[b])
                vs[a], vs[b] = lo, hi
            for t in range(NQ):
                tout[t, pl.ds(c0, LN)] = vs[L - 1 - t]
            return carry

        lax.fori_loop(0, Cc // LN, chunk, 0)
        pltpu.sync_copy(tout, out_hbm.at[wid])

    return run(text_feat)


def _main_body(NB, L, HW, W,
               text_ref, img_ref, posF_ref, vec_ref,
               Wq_ref, bq_ref, Wk_ref, bk_ref, M_ref,
               fW1_ref, fb1_ref, fW2_ref, fb2_ref,
               n2g_ref, n2b_ref, n3g_ref, n3b_ref, png_ref, pnb_ref,
               mW1_ref, mb1_ref, mW2_ref, mb2_ref, mW3_ref, mb3_ref,
               x_ref, pts_ref, g_ref, attn_ref):
    sel = text_ref[...]
    q1 = vec_ref[0:1, :]
    cbias = vec_ref[1:2, :]
    qh = _nn(sel.reshape(NB * NQ, C) + q1,
             Wq_ref[...]) + bq_ref[...]

    kin = (img_ref[...] + posF_ref[...][None]).reshape(NB * HW, C)
    kh = _nn(kin, Wk_ref[...]) + bk_ref[...]
    colh = jax.lax.broadcasted_iota(jnp.int32, (1, C), 1) // DH
    ss = []
    for i in range(NB):
        qh_i = qh[i * NQ:(i + 1) * NQ, :]
        qblk = jnp.concatenate(
            [jnp.where(colh == h, qh_i, 0.0) for h in range(NH)], axis=0)
        ss.append(_nt(qblk, kh[i * HW:(i + 1) * HW, :]))
    s = jnp.concatenate(ss, axis=0)
    s = s / math.sqrt(DH)
    p = jax.nn.softmax(s, axis=-1)

    am = jnp.mean(p.reshape(NB, NH, NQ, HW), axis=1)
    g = jnp.max(am, axis=1, keepdims=True)
    g_ref[...] = g
    attn_ref[...] = am

    coli = jax.lax.broadcasted_iota(jnp.int32, (NB, HW), 1)
    cur = g.reshape(NB, HW)
    hots = []
    xs = []
    ys = []
    for _ in range(NQ):
        m = jnp.max(cur, axis=1, keepdims=True)
        idx = jnp.min(jnp.where(cur == m, coli, HW), axis=1, keepdims=True)
        hit = coli == idx
        hots.append(hit.astype(jnp.float32)[:, None, :])
        cur = jnp.where(hit, NEG, cur)
        xs.append((((idx % W).astype(jnp.float32) + 0.5) / W)[:, None, :])
        ys.append((((idx // W).astype(jnp.float32) + 0.5)
                   / (HW // W))[:, None, :])
    pts_ref[...] = jnp.concatenate(
        [jnp.concatenate(xs, axis=1), jnp.concatenate(ys, axis=1)], axis=2)

    oh = jnp.concatenate(hots, axis=1)
    zpad = jnp.zeros((6, HW), jnp.float32)
    ctxs = [_nn(jnp.concatenate(
                [p[i * NH * NQ:(i + 1) * NH * NQ], oh[i], zpad], axis=0),
                img_ref[i]) for i in range(NB)]

    cah = jnp.concatenate(
        [jnp.concatenate([c[h * NQ:(h + 1) * NQ, :] for c in ctxs], axis=0)
         for h in range(NH)], axis=1)
    ca = cbias + _nn(cah, M_ref[...].reshape(NH * C, C))
    q2 = _ln_rows(q1 + ca, n2g_ref[...], n2b_ref[...])
    ffn = _nn(jnp.maximum(_nn(q2, fW1_ref[...]) + fb1_ref[...], 0.0),
              fW2_ref[...]) + fb2_ref[...]
    q3 = _ln_rows(q2 + ffn, n3g_ref[...], n3b_ref[...])
    q4 = _ln_rows(q3, png_ref[...], pnb_ref[...])

    pos_feat = jnp.concatenate(
        [c[NH * NQ:NH * NQ + NQ, :] for c in ctxs], axis=0)
    x = jnp.concatenate([q4, pos_feat], axis=1)
    x = jnp.maximum(_nn(x, mW1_ref[...]) + mb1_ref[...], 0.0)
    x = jnp.maximum(_nn(x, mW2_ref[...]) + mb2_ref[...], 0.0)
    x = _nn(x, mW3_ref[...]) + mb3_ref[...]
    x_ref[...] = x.reshape(NB, NQ, C)


def kernel(text_feat, text_mask, img_feat, params):
    del text_mask
    B, L, _ = text_feat.shape
    _, _, H, W = img_feat.shape
    HW = H * W
    img = img_feat.reshape(B, C, HW).transpose(0, 2, 1)
    p = params
    r = lambda v: v.reshape(1, -1)

    sel = _sc_text_topk(text_feat)

    posF, M_all, vec = pl.pallas_call(
        functools.partial(_precompute_body, H, W),
        out_shape=[
            jax.ShapeDtypeStruct((HW, C), jnp.float32),
            jax.ShapeDtypeStruct((NH, C, C), jnp.float32),
            jax.ShapeDtypeStruct((8, C), jnp.float32),
        ],
    )(p['sa_Wo'], r(p['sa_bo']), r(p['sa_bv']), r(p['n1_g']), r(p['n1_b']),
      p['ca_Wv'], p['ca_Wo'], r(p['ca_bv']), r(p['ca_bo']))

    NB = 8
    full = lambda shape: pl.BlockSpec(shape, lambda b: (0,) * len(shape))
    perb = lambda shape: pl.BlockSpec((NB,) + shape,
                                      lambda b: (b,) + (0,) * len(shape))
    x, pts, g, am = pl.pallas_call(
        functools.partial(_main_body, NB, L, HW, W),
        grid=(B // NB,),
        in_specs=[
            perb((NQ, C)), perb((HW, C)), full((HW, C)), full((8, C)),
            full((C, C)), full((1, C)), full((C, C)), full((1, C)),
            full((NH, C, C)),
            full((C, FF)), full((1, FF)), full((FF, C)), full((1, C)),
            full((1, C)), full((1, C)), full((1, C)), full((1, C)),
            full((1, C)), full((1, C)),
            full((2 * C, C)), full((1, C)), full((C, C)), full((1, C)),
            full((C, C)), full((1, C)),
        ],
        out_specs=[perb((NQ, C)), perb((NQ, 2)), perb((1, HW)),
                   perb((NQ, HW))],
        out_shape=[
            jax.ShapeDtypeStruct((B, NQ, C), jnp.float32),
            jax.ShapeDtypeStruct((B, NQ, 2), jnp.float32),
            jax.ShapeDtypeStruct((B, 1, HW), jnp.float32),
            jax.ShapeDtypeStruct((B, NQ, HW), jnp.float32),
        ],
    )(sel, img, posF, vec,
      p['ca_Wq'], r(p['ca_bq']), p['ca_Wk'], r(p['ca_bk']), M_all,
      p['ffn_W1'], r(p['ffn_b1']), p['ffn_W2'], r(p['ffn_b2']),
      r(p['n2_g']), r(p['n2_b']), r(p['n3_g']), r(p['n3_b']),
      r(p['pn_g']), r(p['pn_b']),
      p['mlp_W1'], r(p['mlp_b1']), p['mlp_W2'], r(p['mlp_b2']),
      p['mlp_W3'], r(p['mlp_b3']))

    return (x, pts, g.reshape(B, H, W), am.reshape(B, NQ, H, W))


# speedup vs baseline: 2.0885x; 1.0982x over previous
import functools
import math

import jax
import jax.numpy as jnp
from jax import lax
from jax.experimental import pallas as pl
from jax.experimental.pallas import tpu as pltpu
from jax.experimental.pallas import tpu_sc as plsc

C = 256
NQ = 10
NH = 8
DH = C // NH
FF = 512
NEG = float("-inf")


def _ln_rows(x, g, b):
    m = jnp.mean(x, axis=-1, keepdims=True)
    v = jnp.mean((x - m) ** 2, axis=-1, keepdims=True)
    return (x - m) / jnp.sqrt(v + 1e-5) * g + b


def _nn(a, b):
    return jax.lax.dot_general(a, b, (((1,), (0,)), ((), ())),
                               preferred_element_type=jnp.float32)


def _nt(a, b):
    return jax.lax.dot_general(a, b, (((1,), (1,)), ((), ())),
                               preferred_element_type=jnp.float32)


def _precompute_body(H, W, sa_Wo, sa_bo, sa_bv, n1g, n1b,
                     ca_Wv, ca_Wo, ca_bv, ca_bo,
                     posF_ref, M_ref, vec_ref):
    HW = H * W
    HC = C // 2
    ck = jax.lax.broadcasted_iota(jnp.int32, (H, HC), 1)
    cc = jax.lax.broadcasted_iota(jnp.int32, (H, HC), 0).astype(jnp.float32)
    scale = 2.0 * math.pi
    tw = jnp.exp((ck // 2).astype(jnp.float32)
                 * (2.0 / HC) * math.log(10000.0))
    yval = ((cc + 1.0) / (H + 1e-6) * scale) / tw
    xval = ((cc + 1.0) / (W + 1e-6) * scale) / tw
    ytab = jnp.where(ck % 2 == 0, jnp.sin(yval), jnp.cos(yval))
    xtab = jnp.where(ck % 2 == 0, jnp.sin(xval), jnp.cos(xval))
    yexp = jnp.broadcast_to(ytab[:, None, :], (H, W, HC)).reshape(HW, HC)
    xexp = jnp.broadcast_to(xtab[None, :, :], (H, W, HC)).reshape(HW, HC)
    posF_ref[...] = jnp.concatenate([yexp, xexp], axis=1)
    for h in range(NH):
        M_ref[h] = _nn(ca_Wv[:, h * DH:(h + 1) * DH],
                       ca_Wo[h * DH:(h + 1) * DH, :])
    c0 = _nn(sa_bv[...], sa_Wo[...]) + sa_bo[...]
    q1 = _ln_rows(c0, n1g[...], n1b[...])
    cb = _nn(ca_bv[...], ca_Wo[...]) + ca_bo[...]
    vec_ref[...] = jnp.concatenate(
        [q1, cb, jnp.zeros((6, C), jnp.float32)], axis=0)


def _oddeven_merge(lo, n, r):
    step = r * 2
    if step < n:
        yield from _oddeven_merge(lo, n, step)
        yield from _oddeven_merge(lo + r, n, step)
        for i in range(lo + r, lo + n - r, step):
            yield (i, i + r)
    else:
        yield (lo, lo + r)


def _oddeven_sort_pairs(lo, n):
    if n > 1:
        m = n // 2
        yield from _oddeven_sort_pairs(lo, m)
        yield from _oddeven_sort_pairs(lo + m, m)
        yield from _oddeven_merge(lo, n, 1)


def _pruned_sort_pairs(n_real, n_pad):
    return [(i, j) for (i, j) in _oddeven_sort_pairs(0, n_pad)
            if i < n_real and j < n_real]


def _sc_text_topk(text_feat):
    B, L, Cc = text_feat.shape
    info = plsc.get_sparse_core_info()
    ncores, nsub, LN = info.num_cores, info.num_subcores, info.num_lanes
    assert B == ncores * nsub and Cc % LN == 0
    mesh = plsc.VectorSubcoreMesh(core_axis_name="c", subcore_axis_name="s")

    @functools.partial(
        pl.kernel, mesh=mesh,
        out_type=jax.ShapeDtypeStruct((B, NQ, Cc), jnp.float32),
        scratch_types=[
            pltpu.VMEM((L, Cc), jnp.float32),
            pltpu.VMEM((NQ, Cc), jnp.float32),
        ],
    )
    def run(text_hbm, out_hbm, tin, tout):
        wid = lax.axis_index("s") * ncores + lax.axis_index("c")
        pltpu.sync_copy(text_hbm.at[wid], tin)

        npad = 1 << (L - 1).bit_length()
        pairs = _pruned_sort_pairs(L, npad)

        def chunk(ci, carry):
            c0 = ci * LN
            vs = [tin[l, pl.ds(c0, LN)] for l in range(L)]
            for (a, b) in pairs:
                lo = jnp.minimum(vs[a], vs[b])
                hi = jnp.maximum(vs[a], vs[b])
                vs[a], vs[b] = lo, hi
            for t in range(NQ):
                tout[t, pl.ds(c0, LN)] = vs[L - 1 - t]
            return carry

        lax.fori_loop(0, Cc // LN, chunk, 0)
        pltpu.sync_copy(tout, out_hbm.at[wid])

    return run(text_feat)


def _main_body(NB, L, HW, W,
               text_ref, img_ref, posF_ref, vec_ref,
               Wq_ref, bq_ref, Wk_ref, bk_ref, M_ref,
               fW1_ref, fb1_ref, fW2_ref, fb2_ref,
               n2g_ref, n2b_ref, n3g_ref, n3b_ref, png_ref, pnb_ref,
               mW1_ref, mb1_ref, mW2_ref, mb2_ref, mW3_ref, mb3_ref,
               x_ref, pts_ref, g_ref, attn_ref):
    sel = text_ref[...]
    q1 = vec_ref[0:1, :]
    cbias = vec_ref[1:2, :]
    qh = _nn(sel.reshape(NB * NQ, C) + q1,
             Wq_ref[...]) + bq_ref[...]

    kin = (img_ref[...] + posF_ref[...][None]).reshape(NB * HW, C)
    kh = _nn(kin, Wk_ref[...]) + bk_ref[...]
    colh = jax.lax.broadcasted_iota(jnp.int32, (1, C), 1) // DH
    ss = []
    for i in range(NB):
        qh_i = qh[i * NQ:(i + 1) * NQ, :]
        qblk = jnp.concatenate(
            [jnp.where(colh == h, qh_i, 0.0) for h in range(NH)], axis=0)
        ss.append(_nt(qblk, kh[i * HW:(i + 1) * HW, :]))
    s = jnp.concatenate(ss, axis=0)
    s = s / math.sqrt(DH)
    p = jax.nn.softmax(s, axis=-1)

    am = jnp.mean(p.reshape(NB, NH, NQ, HW), axis=1)
    g = jnp.max(am, axis=1, keepdims=True)
    g_ref[...] = g
    attn_ref[...] = am

    coli = jax.lax.broadcasted_iota(jnp.int32, (NB, HW), 1)
    cur = g.reshape(NB, HW)
    hots = []
    xs = []
    ys = []
    for _ in range(NQ):
        m = jnp.max(cur, axis=1, keepdims=True)
        idx = jnp.min(jnp.where(cur == m, coli, HW), axis=1, keepdims=True)
        hit = coli == idx
        hots.append(hit.astype(jnp.float32)[:, None, :])
        cur = jnp.where(hit, NEG, cur)
        xs.append((((idx % W).astype(jnp.float32) + 0.5) / W)[:, None, :])
        ys.append((((idx // W).astype(jnp.float32) + 0.5)
                   / (HW // W))[:, None, :])
    pts_ref[...] = jnp.concatenate(
        [jnp.concatenate(xs, axis=1), jnp.concatenate(ys, axis=1)], axis=2)

    oh = jnp.concatenate(hots, axis=1)
    zpad = jnp.zeros((6, HW), jnp.float32)
    ctxs = [_nn(jnp.concatenate(
                [p[i * NH * NQ:(i + 1) * NH * NQ], oh[i], zpad], axis=0),
                img_ref[i]) for i in range(NB)]

    cah = jnp.concatenate(
        [jnp.concatenate([c[h * NQ:(h + 1) * NQ, :] for c in ctxs], axis=0)
         for h in range(NH)], axis=1)
    ca = cbias + _nn(cah, M_ref[...].reshape(NH * C, C))
    q2 = _ln_rows(q1 + ca, n2g_ref[...], n2b_ref[...])
    ffn = _nn(jnp.maximum(_nn(q2, fW1_ref[...]) + fb1_ref[...], 0.0),
              fW2_ref[...]) + fb2_ref[...]
    q3 = _ln_rows(q2 + ffn, n3g_ref[...], n3b_ref[...])
    q4 = _ln_rows(q3, png_ref[...], pnb_ref[...])

    pos_feat = jnp.concatenate(
        [c[NH * NQ:NH * NQ + NQ, :] for c in ctxs], axis=0)
    x = jnp.concatenate([q4, pos_feat], axis=1)
    x = jnp.maximum(_nn(x, mW1_ref[...]) + mb1_ref[...], 0.0)
    x = jnp.maximum(_nn(x, mW2_ref[...]) + mb2_ref[...], 0.0)
    x = _nn(x, mW3_ref[...]) + mb3_ref[...]
    x_ref[...] = x.reshape(NB, NQ, C)


def kernel(text_feat, text_mask, img_feat, params):
    del text_mask
    B, L, _ = text_feat.shape
    _, _, H, W = img_feat.shape
    HW = H * W
    img = img_feat.reshape(B, C, HW).transpose(0, 2, 1)
    p = params
    r = lambda v: v.reshape(1, -1)

    sel = _sc_text_topk(text_feat)

    posF, M_all, vec = pl.pallas_call(
        functools.partial(_precompute_body, H, W),
        out_shape=[
            jax.ShapeDtypeStruct((HW, C), jnp.float32),
            jax.ShapeDtypeStruct((NH, C, C), jnp.float32),
            jax.ShapeDtypeStruct((8, C), jnp.float32),
        ],
    )(p['sa_Wo'], r(p['sa_bo']), r(p['sa_bv']), r(p['n1_g']), r(p['n1_b']),
      p['ca_Wv'], p['ca_Wo'], r(p['ca_bv']), r(p['ca_bo']))

    NB = 16
    full = lambda shape: pl.BlockSpec(shape, lambda b: (0,) * len(shape))
    perb = lambda shape: pl.BlockSpec((NB,) + shape,
                                      lambda b: (b,) + (0,) * len(shape))
    x, pts, g, am = pl.pallas_call(
        functools.partial(_main_body, NB, L, HW, W),
        grid=(B // NB,),
        in_specs=[
            perb((NQ, C)), perb((HW, C)), full((HW, C)), full((8, C)),
            full((C, C)), full((1, C)), full((C, C)), full((1, C)),
            full((NH, C, C)),
            full((C, FF)), full((1, FF)), full((FF, C)), full((1, C)),
            full((1, C)), full((1, C)), full((1, C)), full((1, C)),
            full((1, C)), full((1, C)),
            full((2 * C, C)), full((1, C)), full((C, C)), full((1, C)),
            full((C, C)), full((1, C)),
        ],
        out_specs=[perb((NQ, C)), perb((NQ, 2)), perb((1, HW)),
                   perb((NQ, HW))],
        out_shape=[
            jax.ShapeDtypeStruct((B, NQ, C), jnp.float32),
            jax.ShapeDtypeStruct((B, NQ, 2), jnp.float32),
            jax.ShapeDtypeStruct((B, 1, HW), jnp.float32),
            jax.ShapeDtypeStruct((B, NQ, HW), jnp.float32),
        ],
    )(sel, img, posF, vec,
      p['ca_Wq'], r(p['ca_bq']), p['ca_Wk'], r(p['ca_bk']), M_all,
      p['ffn_W1'], r(p['ffn_b1']), p['ffn_W2'], r(p['ffn_b2']),
      r(p['n2_g']), r(p['n2_b']), r(p['n3_g']), r(p['n3_b']),
      r(p['pn_g']), r(p['pn_b']),
      p['mlp_W1'], r(p['mlp_b1']), p['mlp_W2'], r(p['mlp_b2']),
      p['mlp_W3'], r(p['mlp_b3']))

    return (x, pts, g.reshape(B, H, W), am.reshape(B, NQ, H, W))
